# Initial kernel scaffold; baseline (speedup 1.0000x reference)
import jax
import jax.numpy as jnp
from jax.experimental import pallas as pl


def kernel(coords, block_type, bt_tile_n_donH, bt_tile_n_acc, bt_tile_donH_inds, bt_tile_acc_inds, bt_tile_donor_type, bt_tile_acceptor_type, pair_params, pair_polynomials, global_params):
    raise NotImplementedError("write your pallas kernel here")


# speedup vs baseline: 202.5102x; 202.5102x over previous
import jax
import jax.numpy as jnp
from jax import lax
from jax.experimental import pallas as pl

P = 16
B = 64
A = 32
T = 32
MAXD = 4
NDT = 8
NAT = 8
NPOLY = 11
ND = B * MAXD
NATOM = B * A

_HI = jax.lax.Precision.HIGHEST


def _dot(a, b):
    return jnp.dot(a, b, precision=_HI, preferred_element_type=jnp.float32)


def _pose_kernel(coords_ref, coordsT_ref, btc_ref, btr_ref, rowtab_ref,
                 coltabT_ref, pflat_ref, ppflat_ref, gp_ref,
                 scores_ref, idx_ref):
    f32 = jnp.float32
    i32 = jnp.int32

    bt_col = btc_ref[0]
    ei = lax.broadcasted_iota(i32, (ND, B), 0)
    eb = lax.broadcasted_iota(i32, (ND, B), 1)
    E = ((ei // MAXD) == eb).astype(f32)
    bt256c = _dot(E, bt_col).astype(i32)

    ri = lax.broadcasted_iota(i32, (ND, T * MAXD), 0)
    rc = lax.broadcasted_iota(i32, (ND, T * MAXD), 1)
    oh128 = ((bt256c == (rc // MAXD)) & ((ri % MAXD) == (rc % MAXD))).astype(f32)
    data = _dot(oh128, rowtab_ref[...])
    don_local = data[:, 0:1].astype(i32)
    ndon = data[:, 1:2].astype(i32)
    oh_dt = data[:, 2:10]

    slot_col = lax.broadcasted_iota(i32, (ND, 1), 0)
    don_atom = don_local + A * (slot_col // MAXD)
    don_mask = (slot_col % MAXD) < ndon

    bt_row = btr_ref[0]
    tb = lax.broadcasted_iota(i32, (B, ND), 0)
    tj = lax.broadcasted_iota(i32, (B, ND), 1)
    ET = ((tj // MAXD) == tb).astype(f32)
    bt256r = _dot(bt_row, ET).astype(i32)

    cr = lax.broadcasted_iota(i32, (T * MAXD, ND), 0)
    cj = lax.broadcasted_iota(i32, (T * MAXD, ND), 1)
    oh128T = ((bt256r == (cr // MAXD)) & ((cj % MAXD) == (cr % MAXD))).astype(f32)
    dataT = _dot(coltabT_ref[...], oh128T)
    acc_local = dataT[0:1, :].astype(i32)
    nacc = dataT[1:2, :].astype(i32)
    oh_atT = dataT[2:10, :]

    slot_row = lax.broadcasted_iota(i32, (1, ND), 1)
    acc_atom = acc_local + A * (slot_row // MAXD)
    acc_mask = (slot_row % MAXD) < nacc

    ai = lax.broadcasted_iota(i32, (ND, NATOM), 1)
    oh_don = (ai == don_atom).astype(f32)
    don_xyz = _dot(oh_don, coords_ref[0])

    aj = lax.broadcasted_iota(i32, (NATOM, ND), 0)
    oh_accT = (aj == acc_atom).astype(f32)
    acc_xyzT = _dot(coordsT_ref[0], oh_accT)

    dx = don_xyz[:, 0:1] - acc_xyzT[0:1, :]
    dy = don_xyz[:, 1:2] - acc_xyzT[1:2, :]
    dz = don_xyz[:, 2:3] - acc_xyzT[2:3, :]
    d2 = ((dx * dx + dy * dy) + dz * dz) + 1e-8
    d = jnp.sqrt(d2)

    mall = _dot(oh_dt, pflat_ref[...])
    val = _dot(mall[:, 0:NAT], oh_atT)
    for k in range(1, NPOLY):
        ck = _dot(mall[:, k * NAT:(k + 1) * NAT], oh_atT)
        val = val * d + ck

    mpp = _dot(oh_dt, ppflat_ref[...])
    pp0 = _dot(mpp[:, 0:NAT], oh_atT)
    pp1 = _dot(mpp[:, NAT:2 * NAT], oh_atT)
    dmin = 0.5 + pp0
    dmax = (dmin + 2.0) + pp1

    mask = don_mask & acc_mask & (d > dmin) & (d < dmax)
    gp = gp_ref[0:1, 0:1]
    energy = jnp.where(mask, val * gp, 0.0)

    s = jnp.sum(energy)
    scores_ref[...] = jnp.full((1, 1, 128), s, dtype=f32)

    m = jnp.min(energy)
    fi = (lax.broadcasted_iota(i32, (ND, ND), 0) * ND
          + lax.broadcasted_iota(i32, (ND, ND), 1)).astype(f32)
    idxf = jnp.min(jnp.where(energy == m, fi, float(ND * ND)))
    idx_ref[...] = jnp.full((1, 1, 128), idxf.astype(i32), dtype=i32)


def kernel(coords, block_type, bt_tile_n_donH, bt_tile_n_acc,
           bt_tile_donH_inds, bt_tile_acc_inds, bt_tile_donor_type,
           bt_tile_acceptor_type, pair_params, pair_polynomials,
           global_params):
    f32 = jnp.float32

    coordsT = jnp.transpose(coords, (0, 2, 1))
    btf = block_type.astype(f32)
    btc = btf[:, :, None]
    btr = btf[:, None, :]

    dl = bt_tile_donH_inds.astype(f32).reshape(T * MAXD, 1)
    ndn = jnp.broadcast_to(bt_tile_n_donH[:, None].astype(f32),
                           (T, MAXD)).reshape(T * MAXD, 1)
    dt1h = jax.nn.one_hot(bt_tile_donor_type.reshape(-1), NDT, dtype=f32)
    rowtab = jnp.concatenate([dl, ndn, dt1h], axis=1)

    al = bt_tile_acc_inds.astype(f32).reshape(T * MAXD, 1)
    nac = jnp.broadcast_to(bt_tile_n_acc[:, None].astype(f32),
                           (T, MAXD)).reshape(T * MAXD, 1)
    at1h = jax.nn.one_hot(bt_tile_acceptor_type.reshape(-1), NAT, dtype=f32)
    coltabT = jnp.concatenate([al, nac, at1h], axis=1).T

    pflat = pair_polynomials.transpose(0, 2, 1).reshape(NDT, NPOLY * NAT)
    ppflat = pair_params.transpose(0, 2, 1)[:, :2, :].reshape(NDT, 2 * NAT)

    scores, idx = pl.pallas_call(
        _pose_kernel,
        grid=(P,),
        in_specs=[
            pl.BlockSpec((1, NATOM, 3), lambda p: (p, 0, 0)),
            pl.BlockSpec((1, 3, NATOM), lambda p: (p, 0, 0)),
            pl.BlockSpec((1, B, 1), lambda p: (p, 0, 0)),
            pl.BlockSpec((1, 1, B), lambda p: (p, 0, 0)),
            pl.BlockSpec((T * MAXD, 10), lambda p: (0, 0)),
            pl.BlockSpec((10, T * MAXD), lambda p: (0, 0)),
            pl.BlockSpec((NDT, NPOLY * NAT), lambda p: (0, 0)),
            pl.BlockSpec((NDT, 2 * NAT), lambda p: (0, 0)),
            pl.BlockSpec((1, 5), lambda p: (0, 0)),
        ],
        out_specs=[
            pl.BlockSpec((1, 1, 128), lambda p: (p, 0, 0)),
            pl.BlockSpec((1, 1, 128), lambda p: (p, 0, 0)),
        ],
        out_shape=[
            jax.ShapeDtypeStruct((P, 1, 128), f32),
            jax.ShapeDtypeStruct((P, 1, 128), jnp.int32),
        ],
    )(coords, coordsT, btc, btr, rowtab, coltabT, pflat, ppflat,
      global_params)

    return scores[:, 0, 0], idx[:, 0, 0]


# speedup vs baseline: 357.8447x; 1.7670x over previous
import jax
import jax.numpy as jnp
from jax import lax
from jax.experimental import pallas as pl

P = 16
B = 64
A = 32
T = 32
MAXD = 4
NDT = 8
NAT = 8
NPOLY = 11
ND = B * MAXD
NATOM = B * A

_HI = jax.lax.Precision.HIGHEST


def _dot(a, b):
    return jnp.dot(a, b, precision=_HI, preferred_element_type=jnp.float32)


def _dot16(a, b):
    return jnp.dot(a.astype(jnp.bfloat16), b.astype(jnp.bfloat16),
                   preferred_element_type=jnp.float32)


def _pose_kernel(xyz_ref, xyzT_ref, btc_ref, btr_ref, rowtab_ref,
                 coltabT_ref, pflat_ref, ppflat_ref, gp_ref,
                 scores_ref, idx_ref):
    f32 = jnp.float32
    i32 = jnp.int32

    bt_col = btc_ref[0]
    ei = lax.broadcasted_iota(i32, (ND, B), 0)
    eb = lax.broadcasted_iota(i32, (ND, B), 1)
    E = ((ei // MAXD) == eb).astype(f32)
    bt256c = _dot16(E, bt_col).astype(i32)

    ri = lax.broadcasted_iota(i32, (ND, T * MAXD), 0)
    rc = lax.broadcasted_iota(i32, (ND, T * MAXD), 1)
    oh128 = ((bt256c == (rc // MAXD)) & ((ri % MAXD) == (rc % MAXD))).astype(f32)
    data = _dot16(oh128, rowtab_ref[...])
    don_local = data[:, 0:1].astype(i32)
    ndon = data[:, 1:2].astype(i32)
    oh_dt = data[:, 2:10]

    slot_col = lax.broadcasted_iota(i32, (ND, 1), 0)
    don_mask = (slot_col % MAXD) < ndon

    bt_row = btr_ref[0]
    tb = lax.broadcasted_iota(i32, (B, ND), 0)
    tj = lax.broadcasted_iota(i32, (B, ND), 1)
    ET = ((tj // MAXD) == tb).astype(f32)
    bt256r = _dot16(bt_row, ET).astype(i32)

    cr = lax.broadcasted_iota(i32, (T * MAXD, ND), 0)
    cj = lax.broadcasted_iota(i32, (T * MAXD, ND), 1)
    oh128T = ((bt256r == (cr // MAXD)) & ((cj % MAXD) == (cr % MAXD))).astype(f32)
    dataT = _dot16(coltabT_ref[...], oh128T)
    acc_local = dataT[0:1, :].astype(i32)
    nacc = dataT[1:2, :].astype(i32)
    oh_atT = dataT[2:10, :]

    slot_row = lax.broadcasted_iota(i32, (1, ND), 1)
    acc_mask = (slot_row % MAXD) < nacc

    tile = _dot(E, xyz_ref[0])
    cc = lax.broadcasted_iota(i32, (ND, 3 * A), 1) % A
    sel = jnp.where(cc == don_local, tile, 0.0)
    don_x = jnp.sum(sel[:, 0:A], axis=1, keepdims=True)
    don_y = jnp.sum(sel[:, A:2 * A], axis=1, keepdims=True)
    don_z = jnp.sum(sel[:, 2 * A:3 * A], axis=1, keepdims=True)

    tileT = _dot(xyzT_ref[0], ET)
    rr = lax.broadcasted_iota(i32, (3 * A, ND), 0) % A
    selT = jnp.where(rr == acc_local, tileT, 0.0)
    acc_x = jnp.sum(selT[0:A, :], axis=0, keepdims=True)
    acc_y = jnp.sum(selT[A:2 * A, :], axis=0, keepdims=True)
    acc_z = jnp.sum(selT[2 * A:3 * A, :], axis=0, keepdims=True)

    dx = don_x - acc_x
    dy = don_y - acc_y
    dz = don_z - acc_z
    d2 = ((dx * dx + dy * dy) + dz * dz) + 1e-8
    d = jnp.sqrt(d2)

    mall = _dot(oh_dt, pflat_ref[...])
    val = _dot(mall[:, 0:NAT], oh_atT)
    for k in range(1, NPOLY):
        ck = _dot(mall[:, k * NAT:(k + 1) * NAT], oh_atT)
        val = val * d + ck

    mpp = _dot(oh_dt, ppflat_ref[...])
    pp0 = _dot(mpp[:, 0:NAT], oh_atT)
    pp1 = _dot(mpp[:, NAT:2 * NAT], oh_atT)
    dmin = 0.5 + pp0
    dmax = (dmin + 2.0) + pp1

    mask = don_mask & acc_mask & (d > dmin) & (d < dmax)
    gp = gp_ref[0:1, 0:1]
    energy = jnp.where(mask, val * gp, 0.0)

    s = jnp.sum(energy)
    scores_ref[...] = jnp.full((1, 1, 128), s, dtype=f32)

    m = jnp.min(energy)
    fi = (lax.broadcasted_iota(i32, (ND, ND), 0) * ND
          + lax.broadcasted_iota(i32, (ND, ND), 1)).astype(f32)
    idxf = jnp.min(jnp.where(energy == m, fi, float(ND * ND)))
    idx_ref[...] = jnp.full((1, 1, 128), idxf.astype(i32), dtype=i32)


def kernel(coords, block_type, bt_tile_n_donH, bt_tile_n_acc,
           bt_tile_donH_inds, bt_tile_acc_inds, bt_tile_donor_type,
           bt_tile_acceptor_type, pair_params, pair_polynomials,
           global_params):
    f32 = jnp.float32

    xyz = coords.reshape(P, B, A, 3).transpose(0, 1, 3, 2).reshape(P, B, 3 * A)
    xyzT = jnp.transpose(xyz, (0, 2, 1))
    btf = block_type.astype(f32)
    btc = btf[:, :, None]
    btr = btf[:, None, :]

    dl = bt_tile_donH_inds.astype(f32).reshape(T * MAXD, 1)
    ndn = jnp.broadcast_to(bt_tile_n_donH[:, None].astype(f32),
                           (T, MAXD)).reshape(T * MAXD, 1)
    dt1h = jax.nn.one_hot(bt_tile_donor_type.reshape(-1), NDT, dtype=f32)
    rowtab = jnp.concatenate([dl, ndn, dt1h], axis=1)

    al = bt_tile_acc_inds.astype(f32).reshape(T * MAXD, 1)
    nac = jnp.broadcast_to(bt_tile_n_acc[:, None].astype(f32),
                           (T, MAXD)).reshape(T * MAXD, 1)
    at1h = jax.nn.one_hot(bt_tile_acceptor_type.reshape(-1), NAT, dtype=f32)
    coltabT = jnp.concatenate([al, nac, at1h], axis=1).T

    pflat = pair_polynomials.transpose(0, 2, 1).reshape(NDT, NPOLY * NAT)
    ppflat = pair_params.transpose(0, 2, 1)[:, :2, :].reshape(NDT, 2 * NAT)

    scores, idx = pl.pallas_call(
        _pose_kernel,
        grid=(P,),
        in_specs=[
            pl.BlockSpec((1, B, 3 * A), lambda p: (p, 0, 0)),
            pl.BlockSpec((1, 3 * A, B), lambda p: (p, 0, 0)),
            pl.BlockSpec((1, B, 1), lambda p: (p, 0, 0)),
            pl.BlockSpec((1, 1, B), lambda p: (p, 0, 0)),
            pl.BlockSpec((T * MAXD, 10), lambda p: (0, 0)),
            pl.BlockSpec((10, T * MAXD), lambda p: (0, 0)),
            pl.BlockSpec((NDT, NPOLY * NAT), lambda p: (0, 0)),
            pl.BlockSpec((NDT, 2 * NAT), lambda p: (0, 0)),
            pl.BlockSpec((1, 5), lambda p: (0, 0)),
        ],
        out_specs=[
            pl.BlockSpec((1, 1, 128), lambda p: (p, 0, 0)),
            pl.BlockSpec((1, 1, 128), lambda p: (p, 0, 0)),
        ],
        out_shape=[
            jax.ShapeDtypeStruct((P, 1, 128), f32),
            jax.ShapeDtypeStruct((P, 1, 128), jnp.int32),
        ],
    )(xyz, xyzT, btc, btr, rowtab, coltabT, pflat, ppflat,
      global_params)

    return scores[:, 0, 0], idx[:, 0, 0]


# speedup vs baseline: 451.6709x; 1.2622x over previous
import jax
import jax.numpy as jnp
from jax import lax
from jax.experimental import pallas as pl

P = 16
B = 64
A = 32
T = 32
MAXD = 4
NDT = 8
NAT = 8
NPOLY = 11
ND = B * MAXD
NATOM = B * A

def _dot16(a, b):
    return jnp.dot(a.astype(jnp.bfloat16), b.astype(jnp.bfloat16),
                   preferred_element_type=jnp.float32)


def _pose_kernel(xyz_ref, xyzT_ref, btc_ref, btr_ref, rowtab_ref,
                 coltabT_ref, ptab3_ref, gp_ref,
                 scores_ref, idx_ref):
    f32 = jnp.float32
    i32 = jnp.int32

    bt_col = btc_ref[0]
    ei = lax.broadcasted_iota(i32, (ND, B), 0)
    eb = lax.broadcasted_iota(i32, (ND, B), 1)
    E = ((ei // MAXD) == eb).astype(f32)
    bt256c = _dot16(E, bt_col).astype(i32)

    ri = lax.broadcasted_iota(i32, (ND, T * MAXD), 0)
    rc = lax.broadcasted_iota(i32, (ND, T * MAXD), 1)
    oh128 = ((bt256c == (rc // MAXD)) & ((ri % MAXD) == (rc % MAXD))).astype(f32)
    data = _dot16(oh128, rowtab_ref[...])
    don_local = data[:, 0:1].astype(i32)
    ndon = data[:, 1:2].astype(i32)
    oh_dt = data[:, 2:10]

    slot_col = lax.broadcasted_iota(i32, (ND, 1), 0)
    don_mask = (slot_col % MAXD) < ndon

    bt_row = btr_ref[0]
    tb = lax.broadcasted_iota(i32, (B, ND), 0)
    tj = lax.broadcasted_iota(i32, (B, ND), 1)
    ET = ((tj // MAXD) == tb).astype(f32)
    bt256r = _dot16(bt_row, ET).astype(i32)

    cr = lax.broadcasted_iota(i32, (T * MAXD, ND), 0)
    cj = lax.broadcasted_iota(i32, (T * MAXD, ND), 1)
    oh128T = ((bt256r == (cr // MAXD)) & ((cj % MAXD) == (cr % MAXD))).astype(f32)
    dataT = _dot16(coltabT_ref[...], oh128T)
    acc_local = dataT[0:1, :].astype(i32)
    nacc = dataT[1:2, :].astype(i32)
    oh_atT = dataT[2:10, :]

    slot_row = lax.broadcasted_iota(i32, (1, ND), 1)
    acc_mask = (slot_row % MAXD) < nacc

    M9 = _dot16(E, xyz_ref[0])
    tile = (M9[:, 0:3 * A] + M9[:, 3 * A:6 * A]) + M9[:, 6 * A:9 * A]
    cc = lax.broadcasted_iota(i32, (ND, 3 * A), 1) % A
    sel = jnp.where(cc == don_local, tile, 0.0)
    don_x = jnp.sum(sel[:, 0:A], axis=1, keepdims=True)
    don_y = jnp.sum(sel[:, A:2 * A], axis=1, keepdims=True)
    don_z = jnp.sum(sel[:, 2 * A:3 * A], axis=1, keepdims=True)

    S9 = _dot16(xyzT_ref[0], ET)
    tileT = (S9[0:3 * A, :] + S9[3 * A:6 * A, :]) + S9[6 * A:9 * A, :]
    rr = lax.broadcasted_iota(i32, (3 * A, ND), 0) % A
    selT = jnp.where(rr == acc_local, tileT, 0.0)
    acc_x = jnp.sum(selT[0:A, :], axis=0, keepdims=True)
    acc_y = jnp.sum(selT[A:2 * A, :], axis=0, keepdims=True)
    acc_z = jnp.sum(selT[2 * A:3 * A, :], axis=0, keepdims=True)

    dx = don_x - acc_x
    dy = don_y - acc_y
    dz = don_z - acc_z
    d2 = ((dx * dx + dy * dy) + dz * dz) + 1e-8
    d = jnp.sqrt(d2)

    mall3 = _dot16(oh_dt, ptab3_ref[...])
    W = 3 * NAT

    def coeff(k):
        base = k * W
        L = jnp.concatenate(
            [mall3[:, base:base + NAT],
             mall3[:, base + NAT:base + 2 * NAT],
             mall3[:, base + 2 * NAT:base + 3 * NAT]], axis=0)
        S = _dot16(L, oh_atT)
        return (S[0:ND] + S[ND:2 * ND]) + S[2 * ND:3 * ND]

    val = coeff(0)
    for k in range(1, NPOLY):
        val = val * d + coeff(k)

    pp0 = coeff(NPOLY)
    pp1 = coeff(NPOLY + 1)
    dmin = 0.5 + pp0
    dmax = (dmin + 2.0) + pp1

    mask = don_mask & acc_mask & (d > dmin) & (d < dmax)
    gp = gp_ref[0:1, 0:1]
    energy = jnp.where(mask, val * gp, 0.0)

    s = jnp.sum(energy)
    scores_ref[...] = jnp.full((1, 1, 128), s, dtype=f32)

    m = jnp.min(energy)
    fi = (lax.broadcasted_iota(i32, (ND, ND), 0) * ND
          + lax.broadcasted_iota(i32, (ND, ND), 1)).astype(f32)
    idxf = jnp.min(jnp.where(energy == m, fi, float(ND * ND)))
    idx_ref[...] = jnp.full((1, 1, 128), idxf.astype(i32), dtype=i32)


def kernel(coords, block_type, bt_tile_n_donH, bt_tile_n_acc,
           bt_tile_donH_inds, bt_tile_acc_inds, bt_tile_donor_type,
           bt_tile_acceptor_type, pair_params, pair_polynomials,
           global_params):
    f32 = jnp.float32

    def split3(x):
        hi = jax.lax.reduce_precision(x, 8, 7)
        r = x - hi
        mid = jax.lax.reduce_precision(r, 8, 7)
        lo = r - mid
        return hi, mid, lo

    xyz = coords.reshape(P, B, A, 3).transpose(0, 1, 3, 2).reshape(P, B, 3 * A)
    xyz9 = jnp.concatenate(split3(xyz), axis=2)
    xyzT9 = jnp.transpose(xyz9, (0, 2, 1))
    btf = block_type.astype(f32)
    btc = btf[:, :, None]
    btr = btf[:, None, :]

    dl = bt_tile_donH_inds.astype(f32).reshape(T * MAXD, 1)
    ndn = jnp.broadcast_to(bt_tile_n_donH[:, None].astype(f32),
                           (T, MAXD)).reshape(T * MAXD, 1)
    dt1h = jax.nn.one_hot(bt_tile_donor_type.reshape(-1), NDT, dtype=f32)
    rowtab = jnp.concatenate([dl, ndn, dt1h], axis=1)

    al = bt_tile_acc_inds.astype(f32).reshape(T * MAXD, 1)
    nac = jnp.broadcast_to(bt_tile_n_acc[:, None].astype(f32),
                           (T, MAXD)).reshape(T * MAXD, 1)
    at1h = jax.nn.one_hot(bt_tile_acceptor_type.reshape(-1), NAT, dtype=f32)
    coltabT = jnp.concatenate([al, nac, at1h], axis=1).T

    pflat = pair_polynomials.transpose(0, 2, 1).reshape(NDT, NPOLY, NAT)
    ppflat = pair_params.transpose(0, 2, 1)[:, :2, :]
    tab = jnp.concatenate([pflat, ppflat], axis=1)
    ptab3 = jnp.concatenate(split3(tab), axis=2).reshape(NDT, 13 * 3 * NAT)

    scores, idx = pl.pallas_call(
        _pose_kernel,
        grid=(P,),
        in_specs=[
            pl.BlockSpec((1, B, 9 * A), lambda p: (p, 0, 0)),
            pl.BlockSpec((1, 9 * A, B), lambda p: (p, 0, 0)),
            pl.BlockSpec((1, B, 1), lambda p: (p, 0, 0)),
            pl.BlockSpec((1, 1, B), lambda p: (p, 0, 0)),
            pl.BlockSpec((T * MAXD, 10), lambda p: (0, 0)),
            pl.BlockSpec((10, T * MAXD), lambda p: (0, 0)),
            pl.BlockSpec((NDT, 13 * 3 * NAT), lambda p: (0, 0)),
            pl.BlockSpec((1, 5), lambda p: (0, 0)),
        ],
        out_specs=[
            pl.BlockSpec((1, 1, 128), lambda p: (p, 0, 0)),
            pl.BlockSpec((1, 1, 128), lambda p: (p, 0, 0)),
        ],
        out_shape=[
            jax.ShapeDtypeStruct((P, 1, 128), f32),
            jax.ShapeDtypeStruct((P, 1, 128), jnp.int32),
        ],
    )(xyz9, xyzT9, btc, btr, rowtab, coltabT, ptab3, global_params)

    return scores[:, 0, 0], idx[:, 0, 0]


# speedup vs baseline: 476.2033x; 1.0543x over previous
import jax
import jax.numpy as jnp
from jax import lax
from jax.experimental import pallas as pl

P = 16
B = 64
A = 32
T = 32
MAXD = 4
NDT = 8
NAT = 8
NPOLY = 11
ND = B * MAXD
NATOM = B * A

def _dot16(a, b):
    return jnp.dot(a.astype(jnp.bfloat16), b.astype(jnp.bfloat16),
                   preferred_element_type=jnp.float32)


def _pose_kernel(xyz_ref, xyzT_ref, btc_ref, btr_ref, rowtab_ref,
                 coltabT_ref, ptab3_ref, gp_ref,
                 scores_ref, idx_ref):
    f32 = jnp.float32
    i32 = jnp.int32

    bt_col = btc_ref[0]
    ei = lax.broadcasted_iota(i32, (ND, B), 0)
    eb = lax.broadcasted_iota(i32, (ND, B), 1)
    E = ((ei // MAXD) == eb).astype(f32)
    bt256c = _dot16(E, bt_col).astype(i32)

    ri = lax.broadcasted_iota(i32, (ND, T * MAXD), 0)
    rc = lax.broadcasted_iota(i32, (ND, T * MAXD), 1)
    oh128 = ((bt256c == (rc // MAXD)) & ((ri % MAXD) == (rc % MAXD))).astype(f32)
    data = _dot16(oh128, rowtab_ref[...])
    don_local = data[:, 0:1].astype(i32)
    ndon = data[:, 1:2].astype(i32)
    oh_dt = data[:, 2:10]

    slot_col = lax.broadcasted_iota(i32, (ND, 1), 0)
    don_mask = (slot_col % MAXD) < ndon

    bt_row = btr_ref[0]
    tb = lax.broadcasted_iota(i32, (B, ND), 0)
    tj = lax.broadcasted_iota(i32, (B, ND), 1)
    ET = ((tj // MAXD) == tb).astype(f32)
    bt256r = _dot16(bt_row, ET).astype(i32)

    cr = lax.broadcasted_iota(i32, (T * MAXD, ND), 0)
    cj = lax.broadcasted_iota(i32, (T * MAXD, ND), 1)
    oh128T = ((bt256r == (cr // MAXD)) & ((cj % MAXD) == (cr % MAXD))).astype(f32)
    dataT = _dot16(coltabT_ref[...], oh128T)
    acc_local = dataT[0:1, :].astype(i32)
    nacc = dataT[1:2, :].astype(i32)
    oh_atT = dataT[2:10, :]

    slot_row = lax.broadcasted_iota(i32, (1, ND), 1)
    acc_mask = (slot_row % MAXD) < nacc

    M9 = _dot16(E, xyz_ref[0])
    cc = lax.broadcasted_iota(i32, (ND, 9 * A), 1) % A
    sel9 = jnp.where(cc == don_local, M9, 0.0)
    gr = lax.broadcasted_iota(i32, (9 * A, 9), 0)
    gc = lax.broadcasted_iota(i32, (9 * A, 9), 1)
    G9 = (gc == ((gr // (3 * A)) * 3 + (gr % (3 * A)) // A)).astype(f32)
    don9 = _dot16(sel9, G9)
    don_x = (don9[:, 0:1] + don9[:, 3:4]) + don9[:, 6:7]
    don_y = (don9[:, 1:2] + don9[:, 4:5]) + don9[:, 7:8]
    don_z = (don9[:, 2:3] + don9[:, 5:6]) + don9[:, 8:9]

    S9 = _dot16(xyzT_ref[0], ET)
    rr = lax.broadcasted_iota(i32, (9 * A, ND), 0) % A
    selT9 = jnp.where(rr == acc_local, S9, 0.0)
    jr = lax.broadcasted_iota(i32, (9, 9 * A), 0)
    jc = lax.broadcasted_iota(i32, (9, 9 * A), 1)
    G9T = (jr == ((jc // (3 * A)) * 3 + (jc % (3 * A)) // A)).astype(f32)
    acc9 = _dot16(G9T, selT9)
    acc_x = (acc9[0:1, :] + acc9[3:4, :]) + acc9[6:7, :]
    acc_y = (acc9[1:2, :] + acc9[4:5, :]) + acc9[7:8, :]
    acc_z = (acc9[2:3, :] + acc9[5:6, :]) + acc9[8:9, :]

    dx = don_x - acc_x
    dy = don_y - acc_y
    dz = don_z - acc_z
    d2 = ((dx * dx + dy * dy) + dz * dz) + 1e-8
    d = jnp.sqrt(d2)

    mall3 = _dot16(oh_dt, ptab3_ref[...])
    NK = 13 * NAT
    L_all = jnp.concatenate(
        [mall3[:, 0:NK], mall3[:, NK:2 * NK], mall3[:, 2 * NK:3 * NK]],
        axis=0)

    def coeff(k):
        S = _dot16(L_all[:, k * NAT:(k + 1) * NAT], oh_atT)
        return (S[0:ND] + S[ND:2 * ND]) + S[2 * ND:3 * ND]

    val = coeff(0)
    for k in range(1, NPOLY):
        val = val * d + coeff(k)

    pp0 = coeff(NPOLY)
    pp1 = coeff(NPOLY + 1)
    dmin = 0.5 + pp0
    dmax = (dmin + 2.0) + pp1

    mask = don_mask & acc_mask & (d > dmin) & (d < dmax)
    gp = gp_ref[0:1, 0:1]
    energy = jnp.where(mask, val * gp, 0.0)

    s = jnp.sum(energy)
    scores_ref[...] = jnp.full((1, 1, 128), s, dtype=f32)

    m = jnp.min(energy)
    fi = (lax.broadcasted_iota(i32, (ND, ND), 0) * ND
          + lax.broadcasted_iota(i32, (ND, ND), 1)).astype(f32)
    idxf = jnp.min(jnp.where(energy == m, fi, float(ND * ND)))
    idx_ref[...] = jnp.full((1, 1, 128), idxf.astype(i32), dtype=i32)


def kernel(coords, block_type, bt_tile_n_donH, bt_tile_n_acc,
           bt_tile_donH_inds, bt_tile_acc_inds, bt_tile_donor_type,
           bt_tile_acceptor_type, pair_params, pair_polynomials,
           global_params):
    f32 = jnp.float32

    def split3(x):
        hi = jax.lax.reduce_precision(x, 8, 7)
        r = x - hi
        mid = jax.lax.reduce_precision(r, 8, 7)
        lo = r - mid
        return hi, mid, lo

    xyz = coords.reshape(P, B, A, 3).transpose(0, 1, 3, 2).reshape(P, B, 3 * A)
    xyz9 = jnp.concatenate(split3(xyz), axis=2)
    xyzT9 = jnp.transpose(xyz9, (0, 2, 1))
    btf = block_type.astype(f32)
    btc = btf[:, :, None]
    btr = btf[:, None, :]

    dl = bt_tile_donH_inds.astype(f32).reshape(T * MAXD, 1)
    ndn = jnp.broadcast_to(bt_tile_n_donH[:, None].astype(f32),
                           (T, MAXD)).reshape(T * MAXD, 1)
    dt1h = jax.nn.one_hot(bt_tile_donor_type.reshape(-1), NDT, dtype=f32)
    rowtab = jnp.concatenate([dl, ndn, dt1h], axis=1)

    al = bt_tile_acc_inds.astype(f32).reshape(T * MAXD, 1)
    nac = jnp.broadcast_to(bt_tile_n_acc[:, None].astype(f32),
                           (T, MAXD)).reshape(T * MAXD, 1)
    at1h = jax.nn.one_hot(bt_tile_acceptor_type.reshape(-1), NAT, dtype=f32)
    coltabT = jnp.concatenate([al, nac, at1h], axis=1).T

    pflat = pair_polynomials.transpose(0, 2, 1).reshape(NDT, NPOLY, NAT)
    ppflat = pair_params.transpose(0, 2, 1)[:, :2, :]
    tab = jnp.concatenate([pflat, ppflat], axis=1)
    ptab3 = jnp.concatenate([t.reshape(NDT, 13 * NAT) for t in split3(tab)],
                            axis=1)

    scores, idx = pl.pallas_call(
        _pose_kernel,
        grid=(P,),
        in_specs=[
            pl.BlockSpec((1, B, 9 * A), lambda p: (p, 0, 0)),
            pl.BlockSpec((1, 9 * A, B), lambda p: (p, 0, 0)),
            pl.BlockSpec((1, B, 1), lambda p: (p, 0, 0)),
            pl.BlockSpec((1, 1, B), lambda p: (p, 0, 0)),
            pl.BlockSpec((T * MAXD, 10), lambda p: (0, 0)),
            pl.BlockSpec((10, T * MAXD), lambda p: (0, 0)),
            pl.BlockSpec((NDT, 13 * 3 * NAT), lambda p: (0, 0)),
            pl.BlockSpec((1, 5), lambda p: (0, 0)),
        ],
        out_specs=[
            pl.BlockSpec((1, 1, 128), lambda p: (p, 0, 0)),
            pl.BlockSpec((1, 1, 128), lambda p: (p, 0, 0)),
        ],
        out_shape=[
            jax.ShapeDtypeStruct((P, 1, 128), f32),
            jax.ShapeDtypeStruct((P, 1, 128), jnp.int32),
        ],
    )(xyz9, xyzT9, btc, btr, rowtab, coltabT, ptab3, global_params)

    return scores[:, 0, 0], idx[:, 0, 0]


# speedup vs baseline: 476.8828x; 1.0014x over previous
import jax
import jax.numpy as jnp
from jax import lax
from jax.experimental import pallas as pl
from jax.experimental.pallas import tpu as pltpu

P = 16
B = 64
A = 32
T = 32
MAXD = 4
NDT = 8
NAT = 8
NPOLY = 11
ND = B * MAXD
NATOM = B * A

def _dot16(a, b):
    return jnp.dot(a.astype(jnp.bfloat16), b.astype(jnp.bfloat16),
                   preferred_element_type=jnp.float32)


def _pose_kernel(xyz_ref, xyzT_ref, btc_ref, btr_ref, rowtab_ref,
                 coltabT_ref, ptab3_ref, gp_ref,
                 scores_ref, idx_ref):
    f32 = jnp.float32
    i32 = jnp.int32

    bt_col = btc_ref[0]
    ei = lax.broadcasted_iota(i32, (ND, B), 0)
    eb = lax.broadcasted_iota(i32, (ND, B), 1)
    E = ((ei // MAXD) == eb).astype(f32)
    bt256c = _dot16(E, bt_col).astype(i32)

    ri = lax.broadcasted_iota(i32, (ND, T * MAXD), 0)
    rc = lax.broadcasted_iota(i32, (ND, T * MAXD), 1)
    oh128 = ((bt256c == (rc // MAXD)) & ((ri % MAXD) == (rc % MAXD))).astype(f32)
    data = _dot16(oh128, rowtab_ref[...])
    don_local = data[:, 0:1].astype(i32)
    ndon = data[:, 1:2].astype(i32)
    oh_dt = data[:, 2:10]

    slot_col = lax.broadcasted_iota(i32, (ND, 1), 0)
    don_mask = (slot_col % MAXD) < ndon

    bt_row = btr_ref[0]
    tb = lax.broadcasted_iota(i32, (B, ND), 0)
    tj = lax.broadcasted_iota(i32, (B, ND), 1)
    ET = ((tj // MAXD) == tb).astype(f32)
    bt256r = _dot16(bt_row, ET).astype(i32)

    cr = lax.broadcasted_iota(i32, (T * MAXD, ND), 0)
    cj = lax.broadcasted_iota(i32, (T * MAXD, ND), 1)
    oh128T = ((bt256r == (cr // MAXD)) & ((cj % MAXD) == (cr % MAXD))).astype(f32)
    dataT = _dot16(coltabT_ref[...], oh128T)
    acc_local = dataT[0:1, :].astype(i32)
    nacc = dataT[1:2, :].astype(i32)
    oh_atT = dataT[2:10, :]

    slot_row = lax.broadcasted_iota(i32, (1, ND), 1)
    acc_mask = (slot_row % MAXD) < nacc

    M9 = _dot16(E, xyz_ref[0])
    cc = lax.broadcasted_iota(i32, (ND, 9 * A), 1) % A
    sel9 = jnp.where(cc == don_local, M9, 0.0)
    gr = lax.broadcasted_iota(i32, (9 * A, 9), 0)
    gc = lax.broadcasted_iota(i32, (9 * A, 9), 1)
    G9 = (gc == ((gr // (3 * A)) * 3 + (gr % (3 * A)) // A)).astype(f32)
    don9 = _dot16(sel9, G9)
    don_x = (don9[:, 0:1] + don9[:, 3:4]) + don9[:, 6:7]
    don_y = (don9[:, 1:2] + don9[:, 4:5]) + don9[:, 7:8]
    don_z = (don9[:, 2:3] + don9[:, 5:6]) + don9[:, 8:9]

    S9 = _dot16(xyzT_ref[0], ET)
    rr = lax.broadcasted_iota(i32, (9 * A, ND), 0) % A
    selT9 = jnp.where(rr == acc_local, S9, 0.0)
    jr = lax.broadcasted_iota(i32, (9, 9 * A), 0)
    jc = lax.broadcasted_iota(i32, (9, 9 * A), 1)
    G9T = (jr == ((jc // (3 * A)) * 3 + (jc % (3 * A)) // A)).astype(f32)
    acc9 = _dot16(G9T, selT9)
    acc_x = (acc9[0:1, :] + acc9[3:4, :]) + acc9[6:7, :]
    acc_y = (acc9[1:2, :] + acc9[4:5, :]) + acc9[7:8, :]
    acc_z = (acc9[2:3, :] + acc9[5:6, :]) + acc9[8:9, :]

    dx = don_x - acc_x
    dy = don_y - acc_y
    dz = don_z - acc_z
    d2 = ((dx * dx + dy * dy) + dz * dz) + 1e-8
    d = jnp.sqrt(d2)

    mall3 = _dot16(oh_dt, ptab3_ref[...])
    NK = 13 * NAT
    L_all = jnp.concatenate(
        [mall3[:, 0:NK], mall3[:, NK:2 * NK], mall3[:, 2 * NK:3 * NK]],
        axis=0)

    def coeff(k):
        S = _dot16(L_all[:, k * NAT:(k + 1) * NAT], oh_atT)
        return (S[0:ND] + S[ND:2 * ND]) + S[2 * ND:3 * ND]

    val = coeff(0)
    for k in range(1, NPOLY):
        val = val * d + coeff(k)

    pp0 = coeff(NPOLY)
    pp1 = coeff(NPOLY + 1)
    dmin = 0.5 + pp0
    dmax = (dmin + 2.0) + pp1

    mask = don_mask & acc_mask & (d > dmin) & (d < dmax)
    gp = gp_ref[0:1, 0:1]
    energy = jnp.where(mask, val * gp, 0.0)

    s = jnp.sum(energy)
    scores_ref[...] = jnp.full((1, 1, 128), s, dtype=f32)

    m = jnp.min(energy)
    fi = (lax.broadcasted_iota(i32, (ND, ND), 0) * ND
          + lax.broadcasted_iota(i32, (ND, ND), 1)).astype(f32)
    idxf = jnp.min(jnp.where(energy == m, fi, float(ND * ND)))
    idx_ref[...] = jnp.full((1, 1, 128), idxf.astype(i32), dtype=i32)


def kernel(coords, block_type, bt_tile_n_donH, bt_tile_n_acc,
           bt_tile_donH_inds, bt_tile_acc_inds, bt_tile_donor_type,
           bt_tile_acceptor_type, pair_params, pair_polynomials,
           global_params):
    f32 = jnp.float32

    def split3(x):
        hi = jax.lax.reduce_precision(x, 8, 7)
        r = x - hi
        mid = jax.lax.reduce_precision(r, 8, 7)
        lo = r - mid
        return hi, mid, lo

    xyz = coords.reshape(P, B, A, 3).transpose(0, 1, 3, 2).reshape(P, B, 3 * A)
    xyz9 = jnp.concatenate(split3(xyz), axis=2)
    xyzT9 = jnp.transpose(xyz9, (0, 2, 1))
    btf = block_type.astype(f32)
    btc = btf[:, :, None]
    btr = btf[:, None, :]

    dl = bt_tile_donH_inds.astype(f32).reshape(T * MAXD, 1)
    ndn = jnp.broadcast_to(bt_tile_n_donH[:, None].astype(f32),
                           (T, MAXD)).reshape(T * MAXD, 1)
    dt1h = jax.nn.one_hot(bt_tile_donor_type.reshape(-1), NDT, dtype=f32)
    rowtab = jnp.concatenate([dl, ndn, dt1h], axis=1)

    al = bt_tile_acc_inds.astype(f32).reshape(T * MAXD, 1)
    nac = jnp.broadcast_to(bt_tile_n_acc[:, None].astype(f32),
                           (T, MAXD)).reshape(T * MAXD, 1)
    at1h = jax.nn.one_hot(bt_tile_acceptor_type.reshape(-1), NAT, dtype=f32)
    coltabT = jnp.concatenate([al, nac, at1h], axis=1).T

    pflat = pair_polynomials.transpose(0, 2, 1).reshape(NDT, NPOLY, NAT)
    ppflat = pair_params.transpose(0, 2, 1)[:, :2, :]
    tab = jnp.concatenate([pflat, ppflat], axis=1)
    ptab3 = jnp.concatenate([t.reshape(NDT, 13 * NAT) for t in split3(tab)],
                            axis=1)

    scores, idx = pl.pallas_call(
        _pose_kernel,
        grid=(P,),
        compiler_params=pltpu.CompilerParams(
            dimension_semantics=("parallel",)),
        in_specs=[
            pl.BlockSpec((1, B, 9 * A), lambda p: (p, 0, 0)),
            pl.BlockSpec((1, 9 * A, B), lambda p: (p, 0, 0)),
            pl.BlockSpec((1, B, 1), lambda p: (p, 0, 0)),
            pl.BlockSpec((1, 1, B), lambda p: (p, 0, 0)),
            pl.BlockSpec((T * MAXD, 10), lambda p: (0, 0)),
            pl.BlockSpec((10, T * MAXD), lambda p: (0, 0)),
            pl.BlockSpec((NDT, 13 * 3 * NAT), lambda p: (0, 0)),
            pl.BlockSpec((1, 5), lambda p: (0, 0)),
        ],
        out_specs=[
            pl.BlockSpec((1, 1, 128), lambda p: (p, 0, 0)),
            pl.BlockSpec((1, 1, 128), lambda p: (p, 0, 0)),
        ],
        out_shape=[
            jax.ShapeDtypeStruct((P, 1, 128), f32),
            jax.ShapeDtypeStruct((P, 1, 128), jnp.int32),
        ],
    )(xyz9, xyzT9, btc, btr, rowtab, coltabT, ptab3, global_params)

    return scores[:, 0, 0], idx[:, 0, 0]


# speedup vs baseline: 486.8574x; 1.0209x over previous
import jax
import jax.numpy as jnp
from jax import lax
from jax.experimental import pallas as pl
from jax.experimental.pallas import tpu as pltpu

P = 16
B = 64
A = 32
T = 32
MAXD = 4
NDT = 8
NAT = 8
NPOLY = 11
ND = B * MAXD
NATOM = B * A

def _dot16(a, b):
    return jnp.dot(a.astype(jnp.bfloat16), b.astype(jnp.bfloat16),
                   preferred_element_type=jnp.float32)


def _pose_kernel(xyz_ref, xyzT_ref, btc_ref, btr_ref, rowtab_ref,
                 coltabT_ref, ptab3_ref, gp_ref,
                 scores_ref, idx_ref):
    f32 = jnp.float32
    i32 = jnp.int32

    bt_col = btc_ref[0]
    ei = lax.broadcasted_iota(i32, (ND, B), 0)
    eb = lax.broadcasted_iota(i32, (ND, B), 1)
    E = ((ei // MAXD) == eb).astype(f32)
    bt256c = _dot16(E, bt_col).astype(i32)

    ri = lax.broadcasted_iota(i32, (ND, T * MAXD), 0)
    rc = lax.broadcasted_iota(i32, (ND, T * MAXD), 1)
    oh128 = ((bt256c == (rc // MAXD)) & ((ri % MAXD) == (rc % MAXD))).astype(f32)
    data = _dot16(oh128, rowtab_ref[...])
    don_local = data[:, 0:1].astype(i32)
    ndon = data[:, 1:2].astype(i32)
    oh_dt = data[:, 2:10]

    slot_col = lax.broadcasted_iota(i32, (ND, 1), 0)
    don_mask = (slot_col % MAXD) < ndon

    bt_row = btr_ref[0]
    tb = lax.broadcasted_iota(i32, (B, ND), 0)
    tj = lax.broadcasted_iota(i32, (B, ND), 1)
    ET = ((tj // MAXD) == tb).astype(f32)
    bt256r = _dot16(bt_row, ET).astype(i32)

    cr = lax.broadcasted_iota(i32, (T * MAXD, ND), 0)
    cj = lax.broadcasted_iota(i32, (T * MAXD, ND), 1)
    oh128T = ((bt256r == (cr // MAXD)) & ((cj % MAXD) == (cr % MAXD))).astype(f32)
    dataT = _dot16(coltabT_ref[...], oh128T)
    acc_local = dataT[0:1, :].astype(i32)
    nacc = dataT[1:2, :].astype(i32)
    oh_atT = dataT[2:10, :]

    slot_row = lax.broadcasted_iota(i32, (1, ND), 1)
    acc_mask = (slot_row % MAXD) < nacc

    M9 = _dot16(E, xyz_ref[0])
    cc = lax.broadcasted_iota(i32, (ND, 9 * A), 1) % A
    sel9 = jnp.where(cc == don_local, M9, 0.0)
    gr = lax.broadcasted_iota(i32, (9 * A, 9), 0)
    gc = lax.broadcasted_iota(i32, (9 * A, 9), 1)
    G9 = (gc == ((gr // (3 * A)) * 3 + (gr % (3 * A)) // A)).astype(f32)
    don9 = _dot16(sel9, G9)
    don_x = (don9[:, 0:1] + don9[:, 3:4]) + don9[:, 6:7]
    don_y = (don9[:, 1:2] + don9[:, 4:5]) + don9[:, 7:8]
    don_z = (don9[:, 2:3] + don9[:, 5:6]) + don9[:, 8:9]

    S9 = _dot16(xyzT_ref[0], ET)
    rr = lax.broadcasted_iota(i32, (9 * A, ND), 0) % A
    selT9 = jnp.where(rr == acc_local, S9, 0.0)
    jr = lax.broadcasted_iota(i32, (9, 9 * A), 0)
    jc = lax.broadcasted_iota(i32, (9, 9 * A), 1)
    G9T = (jr == ((jc // (3 * A)) * 3 + (jc % (3 * A)) // A)).astype(f32)
    acc9 = _dot16(G9T, selT9)
    acc_x = (acc9[0:1, :] + acc9[3:4, :]) + acc9[6:7, :]
    acc_y = (acc9[1:2, :] + acc9[4:5, :]) + acc9[7:8, :]
    acc_z = (acc9[2:3, :] + acc9[5:6, :]) + acc9[8:9, :]

    dx = don_x - acc_x
    dy = don_y - acc_y
    dz = don_z - acc_z
    d2 = ((dx * dx + dy * dy) + dz * dz) + 1e-8
    d = jnp.sqrt(d2)

    mall3 = _dot16(oh_dt, ptab3_ref[...])
    NK = 13 * NAT
    L_all = jnp.concatenate(
        [mall3[:, 0:NK], mall3[:, NK:2 * NK], mall3[:, 2 * NK:3 * NK]],
        axis=0)

    def coeff(k):
        S = _dot16(L_all[:, k * NAT:(k + 1) * NAT], oh_atT)
        return (S[0:ND] + S[ND:2 * ND]) + S[2 * ND:3 * ND]

    val = coeff(0)
    for k in range(1, NPOLY):
        val = val * d + coeff(k)

    pp0 = coeff(NPOLY)
    pp1 = coeff(NPOLY + 1)
    dmin = 0.5 + pp0
    dmax = (dmin + 2.0) + pp1

    mask = don_mask & acc_mask & (d > dmin) & (d < dmax)
    gp = gp_ref[0:1, 0:1]
    energy = jnp.where(mask, val * gp, 0.0)

    s = jnp.sum(energy)
    scores_ref[...] = jnp.full((1, 1, 128), s, dtype=f32)

    m = jnp.min(energy)
    fi = (lax.broadcasted_iota(i32, (ND, ND), 0) * ND
          + lax.broadcasted_iota(i32, (ND, ND), 1)).astype(f32)
    idxf = jnp.min(jnp.where(energy == m, fi, float(ND * ND)))
    idx_ref[...] = jnp.full((1, 1, 128), idxf.astype(i32), dtype=i32)


def kernel(coords, block_type, bt_tile_n_donH, bt_tile_n_acc,
           bt_tile_donH_inds, bt_tile_acc_inds, bt_tile_donor_type,
           bt_tile_acceptor_type, pair_params, pair_polynomials,
           global_params):
    f32 = jnp.float32

    def split3(x):
        hi = jax.lax.reduce_precision(x, 8, 7)
        r = x - hi
        mid = jax.lax.reduce_precision(r, 8, 7)
        lo = r - mid
        return hi, mid, lo

    bf16 = jnp.bfloat16
    xyz = coords.reshape(P, B, A, 3).transpose(0, 1, 3, 2).reshape(P, B, 3 * A)
    xyz9 = jnp.concatenate(split3(xyz), axis=2).astype(bf16)
    xyzT9 = jnp.transpose(xyz9, (0, 2, 1))
    btf = block_type.astype(bf16)
    btc = btf[:, :, None]
    btr = btf[:, None, :]

    dl = bt_tile_donH_inds.astype(f32).reshape(T * MAXD, 1)
    ndn = jnp.broadcast_to(bt_tile_n_donH[:, None].astype(f32),
                           (T, MAXD)).reshape(T * MAXD, 1)
    dt1h = jax.nn.one_hot(bt_tile_donor_type.reshape(-1), NDT, dtype=f32)
    rowtab = jnp.concatenate([dl, ndn, dt1h], axis=1).astype(bf16)

    al = bt_tile_acc_inds.astype(f32).reshape(T * MAXD, 1)
    nac = jnp.broadcast_to(bt_tile_n_acc[:, None].astype(f32),
                           (T, MAXD)).reshape(T * MAXD, 1)
    at1h = jax.nn.one_hot(bt_tile_acceptor_type.reshape(-1), NAT, dtype=f32)
    coltabT = jnp.concatenate([al, nac, at1h], axis=1).T.astype(bf16)

    pflat = pair_polynomials.transpose(0, 2, 1).reshape(NDT, NPOLY, NAT)
    ppflat = pair_params.transpose(0, 2, 1)[:, :2, :]
    tab = jnp.concatenate([pflat, ppflat], axis=1)
    ptab3 = jnp.concatenate([t.reshape(NDT, 13 * NAT) for t in split3(tab)],
                            axis=1).astype(bf16)

    scores, idx = pl.pallas_call(
        _pose_kernel,
        grid=(P,),
        compiler_params=pltpu.CompilerParams(
            dimension_semantics=("parallel",)),
        in_specs=[
            pl.BlockSpec((1, B, 9 * A), lambda p: (p, 0, 0)),
            pl.BlockSpec((1, 9 * A, B), lambda p: (p, 0, 0)),
            pl.BlockSpec((1, B, 1), lambda p: (p, 0, 0)),
            pl.BlockSpec((1, 1, B), lambda p: (p, 0, 0)),
            pl.BlockSpec((T * MAXD, 10), lambda p: (0, 0)),
            pl.BlockSpec((10, T * MAXD), lambda p: (0, 0)),
            pl.BlockSpec((NDT, 13 * 3 * NAT), lambda p: (0, 0)),
            pl.BlockSpec((1, 5), lambda p: (0, 0)),
        ],
        out_specs=[
            pl.BlockSpec((1, 1, 128), lambda p: (p, 0, 0)),
            pl.BlockSpec((1, 1, 128), lambda p: (p, 0, 0)),
        ],
        out_shape=[
            jax.ShapeDtypeStruct((P, 1, 128), f32),
            jax.ShapeDtypeStruct((P, 1, 128), jnp.int32),
        ],
    )(xyz9, xyzT9, btc, btr, rowtab, coltabT, ptab3, global_params)

    return scores[:, 0, 0], idx[:, 0, 0]


# speedup vs baseline: 491.2763x; 1.0091x over previous
import jax
import jax.numpy as jnp
from jax import lax
from jax.experimental import pallas as pl
from jax.experimental.pallas import tpu as pltpu

P = 16
B = 64
A = 32
T = 32
MAXD = 4
NDT = 8
NAT = 8
NPOLY = 11
ND = B * MAXD
NATOM = B * A

def _dot16(a, b):
    return jnp.dot(a.astype(jnp.bfloat16), b.astype(jnp.bfloat16),
                   preferred_element_type=jnp.float32)


PPB = 2


def _pose_kernel(xyz_ref, xyzT_ref, btc_ref, btr_ref, rowtab_ref,
                 coltabT_ref, ptab3_ref, gp_ref,
                 scores_ref, idx_ref):
    for sub in range(PPB):
        _one_pose(xyz_ref[sub], xyzT_ref[sub], btc_ref[sub], btr_ref[sub],
                  rowtab_ref, coltabT_ref, ptab3_ref, gp_ref,
                  scores_ref, idx_ref, sub)


def _one_pose(xyz, xyzT, bt_col, bt_row, rowtab_ref,
              coltabT_ref, ptab3_ref, gp_ref, scores_ref, idx_ref, sub):
    f32 = jnp.float32
    i32 = jnp.int32
    ei = lax.broadcasted_iota(i32, (ND, B), 0)
    eb = lax.broadcasted_iota(i32, (ND, B), 1)
    E = ((ei // MAXD) == eb).astype(f32)
    bt256c = _dot16(E, bt_col).astype(i32)

    ri = lax.broadcasted_iota(i32, (ND, T * MAXD), 0)
    rc = lax.broadcasted_iota(i32, (ND, T * MAXD), 1)
    oh128 = ((bt256c == (rc // MAXD)) & ((ri % MAXD) == (rc % MAXD))).astype(f32)
    data = _dot16(oh128, rowtab_ref[...])
    don_local = data[:, 0:1].astype(i32)
    ndon = data[:, 1:2].astype(i32)
    oh_dt = data[:, 2:10]

    slot_col = lax.broadcasted_iota(i32, (ND, 1), 0)
    don_mask = (slot_col % MAXD) < ndon

    tb = lax.broadcasted_iota(i32, (B, ND), 0)
    tj = lax.broadcasted_iota(i32, (B, ND), 1)
    ET = ((tj // MAXD) == tb).astype(f32)
    bt256r = _dot16(bt_row, ET).astype(i32)

    cr = lax.broadcasted_iota(i32, (T * MAXD, ND), 0)
    cj = lax.broadcasted_iota(i32, (T * MAXD, ND), 1)
    oh128T = ((bt256r == (cr // MAXD)) & ((cj % MAXD) == (cr % MAXD))).astype(f32)
    dataT = _dot16(coltabT_ref[...], oh128T)
    acc_local = dataT[0:1, :].astype(i32)
    nacc = dataT[1:2, :].astype(i32)
    oh_atT = dataT[2:10, :]

    slot_row = lax.broadcasted_iota(i32, (1, ND), 1)
    acc_mask = (slot_row % MAXD) < nacc

    M9 = _dot16(E, xyz)
    cc = lax.broadcasted_iota(i32, (ND, 9 * A), 1) % A
    sel9 = jnp.where(cc == don_local, M9, 0.0)
    gr = lax.broadcasted_iota(i32, (9 * A, 9), 0)
    gc = lax.broadcasted_iota(i32, (9 * A, 9), 1)
    G9 = (gc == ((gr // (3 * A)) * 3 + (gr % (3 * A)) // A)).astype(f32)
    don9 = _dot16(sel9, G9)
    don_x = (don9[:, 0:1] + don9[:, 3:4]) + don9[:, 6:7]
    don_y = (don9[:, 1:2] + don9[:, 4:5]) + don9[:, 7:8]
    don_z = (don9[:, 2:3] + don9[:, 5:6]) + don9[:, 8:9]

    S9 = _dot16(xyzT, ET)
    rr = lax.broadcasted_iota(i32, (9 * A, ND), 0) % A
    selT9 = jnp.where(rr == acc_local, S9, 0.0)
    jr = lax.broadcasted_iota(i32, (9, 9 * A), 0)
    jc = lax.broadcasted_iota(i32, (9, 9 * A), 1)
    G9T = (jr == ((jc // (3 * A)) * 3 + (jc % (3 * A)) // A)).astype(f32)
    acc9 = _dot16(G9T, selT9)
    acc_x = (acc9[0:1, :] + acc9[3:4, :]) + acc9[6:7, :]
    acc_y = (acc9[1:2, :] + acc9[4:5, :]) + acc9[7:8, :]
    acc_z = (acc9[2:3, :] + acc9[5:6, :]) + acc9[8:9, :]

    dx = don_x - acc_x
    dy = don_y - acc_y
    dz = don_z - acc_z
    d2 = ((dx * dx + dy * dy) + dz * dz) + 1e-8
    d = jnp.sqrt(d2)

    mall3 = _dot16(oh_dt, ptab3_ref[...])
    NK = 13 * NAT
    L_all = jnp.concatenate(
        [mall3[:, 0:NK], mall3[:, NK:2 * NK], mall3[:, 2 * NK:3 * NK]],
        axis=0)

    def coeff(k):
        S = _dot16(L_all[:, k * NAT:(k + 1) * NAT], oh_atT)
        return (S[0:ND] + S[ND:2 * ND]) + S[2 * ND:3 * ND]

    val = coeff(0)
    for k in range(1, NPOLY):
        val = val * d + coeff(k)

    pp0 = coeff(NPOLY)
    pp1 = coeff(NPOLY + 1)
    dmin = 0.5 + pp0
    dmax = (dmin + 2.0) + pp1

    mask = don_mask & acc_mask & (d > dmin) & (d < dmax)
    gp = gp_ref[0:1, 0:1]
    energy = jnp.where(mask, val * gp, 0.0)

    s = jnp.sum(energy)
    scores_ref[sub] = jnp.full((1, 128), s, dtype=f32)

    m = jnp.min(energy)
    fi = (lax.broadcasted_iota(i32, (ND, ND), 0) * ND
          + lax.broadcasted_iota(i32, (ND, ND), 1)).astype(f32)
    idxf = jnp.min(jnp.where(energy == m, fi, float(ND * ND)))
    idx_ref[sub] = jnp.full((1, 128), idxf.astype(i32), dtype=i32)


def kernel(coords, block_type, bt_tile_n_donH, bt_tile_n_acc,
           bt_tile_donH_inds, bt_tile_acc_inds, bt_tile_donor_type,
           bt_tile_acceptor_type, pair_params, pair_polynomials,
           global_params):
    f32 = jnp.float32

    def split3(x):
        hi = jax.lax.reduce_precision(x, 8, 7)
        r = x - hi
        mid = jax.lax.reduce_precision(r, 8, 7)
        lo = r - mid
        return hi, mid, lo

    bf16 = jnp.bfloat16
    xyz = coords.reshape(P, B, A, 3).transpose(0, 1, 3, 2).reshape(P, B, 3 * A)
    xyz9 = jnp.concatenate(split3(xyz), axis=2).astype(bf16)
    xyzT9 = jnp.transpose(xyz9, (0, 2, 1))
    btf = block_type.astype(bf16)
    btc = btf[:, :, None]
    btr = btf[:, None, :]

    dl = bt_tile_donH_inds.astype(f32).reshape(T * MAXD, 1)
    ndn = jnp.broadcast_to(bt_tile_n_donH[:, None].astype(f32),
                           (T, MAXD)).reshape(T * MAXD, 1)
    dt1h = jax.nn.one_hot(bt_tile_donor_type.reshape(-1), NDT, dtype=f32)
    rowtab = jnp.concatenate([dl, ndn, dt1h], axis=1).astype(bf16)

    al = bt_tile_acc_inds.astype(f32).reshape(T * MAXD, 1)
    nac = jnp.broadcast_to(bt_tile_n_acc[:, None].astype(f32),
                           (T, MAXD)).reshape(T * MAXD, 1)
    at1h = jax.nn.one_hot(bt_tile_acceptor_type.reshape(-1), NAT, dtype=f32)
    coltabT = jnp.concatenate([al, nac, at1h], axis=1).T.astype(bf16)

    pflat = pair_polynomials.transpose(0, 2, 1).reshape(NDT, NPOLY, NAT)
    ppflat = pair_params.transpose(0, 2, 1)[:, :2, :]
    tab = jnp.concatenate([pflat, ppflat], axis=1)
    ptab3 = jnp.concatenate([t.reshape(NDT, 13 * NAT) for t in split3(tab)],
                            axis=1).astype(bf16)

    scores, idx = pl.pallas_call(
        _pose_kernel,
        grid=(P // PPB,),
        compiler_params=pltpu.CompilerParams(
            dimension_semantics=("parallel",)),
        in_specs=[
            pl.BlockSpec((PPB, B, 9 * A), lambda p: (p, 0, 0)),
            pl.BlockSpec((PPB, 9 * A, B), lambda p: (p, 0, 0)),
            pl.BlockSpec((PPB, B, 1), lambda p: (p, 0, 0)),
            pl.BlockSpec((PPB, 1, B), lambda p: (p, 0, 0)),
            pl.BlockSpec((T * MAXD, 10), lambda p: (0, 0)),
            pl.BlockSpec((10, T * MAXD), lambda p: (0, 0)),
            pl.BlockSpec((NDT, 13 * 3 * NAT), lambda p: (0, 0)),
            pl.BlockSpec((1, 5), lambda p: (0, 0)),
        ],
        out_specs=[
            pl.BlockSpec((PPB, 1, 128), lambda p: (p, 0, 0)),
            pl.BlockSpec((PPB, 1, 128), lambda p: (p, 0, 0)),
        ],
        out_shape=[
            jax.ShapeDtypeStruct((P, 1, 128), f32),
            jax.ShapeDtypeStruct((P, 1, 128), jnp.int32),
        ],
    )(xyz9, xyzT9, btc, btr, rowtab, coltabT, ptab3, global_params)

    return scores[:, 0, 0], idx[:, 0, 0]
